# Initial kernel scaffold; baseline (speedup 1.0000x reference)
import jax
import jax.numpy as jnp
from jax.experimental import pallas as pl


def kernel(x, edge_index, edge_attr, W1, b1, W2, b2, W3, b3):
    raise NotImplementedError("write your pallas kernel here")


# speedup vs baseline: 6.7927x; 6.7927x over previous
import functools

import jax
import jax.numpy as jnp
from jax import lax
from jax.experimental import pallas as pl
from jax.experimental.pallas import tpu as pltpu
from jax.experimental.pallas import tpu_sc as plsc

N = 10000
NP = 10240
E = 320000
D = 128

NC = 2
NS = 16
NW = NC * NS
CH = 128
NCHT = 80
E_PAD = NW * NCHT * CH
ROWS_PT = NP // NS

_MESH = plsc.VectorSubcoreMesh(core_axis_name="c", subcore_axis_name="s")


def _deg_body(dstm, ewm, z_hbm, out_hbm, dst_v, ew_v, deg_sh):
    cid = lax.axis_index("c")
    sid = lax.axis_index("s")
    wid = sid * NC + cid
    pltpu.sync_copy(z_hbm.at[pl.ds(sid * ROWS_PT, ROWS_PT)],
                    deg_sh.at[pl.ds(sid * ROWS_PT, ROWS_PT)])
    pltpu.sync_copy(dstm.at[wid], dst_v)
    pltpu.sync_copy(ewm.at[wid], ew_v)
    plsc.subcore_barrier()

    def chunk(j, carry):
        pltpu.sync_copy(ew_v.at[j], deg_sh.at[dst_v.at[j]], add=True)
        return carry

    lax.fori_loop(0, NCHT, chunk, 0)
    plsc.subcore_barrier()
    pltpu.sync_copy(deg_sh.at[pl.ds(sid * ROWS_PT, ROWS_PT)],
                    out_hbm.at[cid, pl.ds(sid * ROWS_PT, ROWS_PT)])


_deg_kernel = pl.kernel(
    _deg_body, mesh=_MESH,
    out_type=jax.ShapeDtypeStruct((NC, NP), jnp.float32),
    scratch_types=[
        pltpu.VMEM((NCHT, CH), jnp.int32),
        pltpu.VMEM((NCHT, CH), jnp.float32),
        pltpu.VMEM_SHARED((NP,), jnp.float32),
    ],
)


def _agg_body(h_hbm, srcm, dstm, ewm, z_hbm, out_hbm,
              src_v, dst_v, ew_v, rows, acc_sh, sem):
    cid = lax.axis_index("c")
    sid = lax.axis_index("s")
    wid = sid * NC + cid
    pltpu.sync_copy(z_hbm.at[pl.ds(sid * ROWS_PT, ROWS_PT)],
                    acc_sh.at[pl.ds(sid * ROWS_PT, ROWS_PT)])
    pltpu.sync_copy(srcm.at[wid], src_v)
    pltpu.sync_copy(dstm.at[wid], dst_v)
    pltpu.sync_copy(ewm.at[wid], ew_v)
    plsc.subcore_barrier()

    def chunk(j, carry):
        pltpu.async_copy(h_hbm.at[src_v.at[j]], rows, sem).wait()

        def edge_group(g, c2):
            ews = ew_v[j, pl.ds(g * 16, 16)]
            for i in range(16):
                s = ews[i]
                e = g * 16 + i
                for k in range(D // 16):
                    sl = pl.ds(k * 16, 16)
                    rows[e, sl] = rows[e, sl] * s
            return c2

        lax.fori_loop(0, CH // 16, edge_group, 0)
        pltpu.sync_copy(rows, acc_sh.at[dst_v.at[j]], add=True)
        return carry

    lax.fori_loop(0, NCHT, chunk, 0)
    plsc.subcore_barrier()
    pltpu.sync_copy(acc_sh.at[pl.ds(sid * ROWS_PT, ROWS_PT)],
                    out_hbm.at[cid, pl.ds(sid * ROWS_PT, ROWS_PT)])


_agg_kernel = pl.kernel(
    _agg_body, mesh=_MESH,
    out_type=jax.ShapeDtypeStruct((NC, NP, D), jnp.float32),
    scratch_types=[
        pltpu.VMEM((NCHT, CH), jnp.int32),
        pltpu.VMEM((NCHT, CH), jnp.int32),
        pltpu.VMEM((NCHT, CH), jnp.float32),
        pltpu.VMEM((CH, D), jnp.float32),
        pltpu.VMEM_SHARED((NP, D), jnp.float32),
        pltpu.SemaphoreType.DMA,
    ],
)


BM = 1024


def _dis(d0_ref, d1_ref):
    return lax.rsqrt(d0_ref[...] + d1_ref[...] + 1.0)


def _mm1_body(x_ref, w_ref, d0_ref, d1_ref, o_ref):
    dis = _dis(d0_ref, d1_ref)
    o_ref[...] = dis * jnp.dot(x_ref[...], w_ref[...],
                               preferred_element_type=jnp.float32)


def _mid_body(a0_ref, a1_ref, hp_ref, d0_ref, d1_ref, b_ref, w_ref, o_ref):
    dis = _dis(d0_ref, d1_ref)
    t = dis * (a0_ref[...] + a1_ref[...] + hp_ref[...]) + b_ref[...]
    t = jnp.where(t >= 0.0, t, 0.01 * t)
    o_ref[...] = dis * jnp.dot(t, w_ref[...],
                               preferred_element_type=jnp.float32)


def _fin_body(a0_ref, a1_ref, hp_ref, d0_ref, d1_ref, b_ref, o_ref):
    dis = _dis(d0_ref, d1_ref)
    o_ref[...] = dis * (a0_ref[...] + a1_ref[...] + hp_ref[...]) + b_ref[...]


_nd_spec = pl.BlockSpec((BM, D), lambda i: (i, 0))
_d_spec = pl.BlockSpec((BM, 1), lambda i: (i, 0))
_w_spec = pl.BlockSpec((D, D), lambda i: (0, 0))
_b_spec = pl.BlockSpec((1, D), lambda i: (0, 0))
_out_nd = jax.ShapeDtypeStruct((NP, D), jnp.float32)
_grid = (NP // BM,)

_mm1 = pl.pallas_call(
    _mm1_body, grid=_grid,
    in_specs=[_nd_spec, _w_spec, _d_spec, _d_spec],
    out_specs=_nd_spec, out_shape=_out_nd)

_mid = pl.pallas_call(
    _mid_body, grid=_grid,
    in_specs=[_nd_spec, _nd_spec, _nd_spec, _d_spec, _d_spec, _b_spec, _w_spec],
    out_specs=_nd_spec, out_shape=_out_nd)

_fin = pl.pallas_call(
    _fin_body, grid=_grid,
    in_specs=[_nd_spec, _nd_spec, _nd_spec, _d_spec, _d_spec, _b_spec],
    out_specs=_nd_spec, out_shape=_out_nd)


def kernel(x, edge_index, edge_attr, W1, b1, W2, b2, W3, b3):
    src = edge_index[0].astype(jnp.int32)
    dst = edge_index[1].astype(jnp.int32)
    ew = edge_attr.astype(jnp.float32)

    pad = E_PAD - E
    srcm = jnp.concatenate([src, jnp.zeros((pad,), jnp.int32)])
    dstm = jnp.concatenate([dst, jnp.zeros((pad,), jnp.int32)])
    ewm = jnp.concatenate([ew, jnp.zeros((pad,), jnp.float32)])
    srcm = srcm.reshape(NW, NCHT, CH)
    dstm = dstm.reshape(NW, NCHT, CH)
    ewm = ewm.reshape(NW, NCHT, CH)

    xpad = jnp.concatenate([x, jnp.zeros((NP - N, D), jnp.float32)])
    z_n1 = jnp.zeros((NP,), jnp.float32)
    z_nd = jnp.zeros((NP, D), jnp.float32)

    degp = _deg_kernel(dstm, ewm, z_n1)
    d0 = degp[0].reshape(NP, 1)
    d1 = degp[1].reshape(NP, 1)

    b1r = b1.reshape(1, D)
    b2r = b2.reshape(1, D)
    b3r = b3.reshape(1, D)

    hp = _mm1(xpad, W1, d0, d1)
    accp = _agg_kernel(hp, srcm, dstm, ewm, z_nd)
    hp = _mid(accp[0], accp[1], hp, d0, d1, b1r, W2)
    accp = _agg_kernel(hp, srcm, dstm, ewm, z_nd)
    hp = _mid(accp[0], accp[1], hp, d0, d1, b2r, W3)
    accp = _agg_kernel(hp, srcm, dstm, ewm, z_nd)
    return _fin(accp[0], accp[1], hp, d0, d1, b3r)[:N]


# speedup vs baseline: 6.7953x; 1.0004x over previous
import functools

import jax
import jax.numpy as jnp
from jax import lax
from jax.experimental import pallas as pl
from jax.experimental.pallas import tpu as pltpu
from jax.experimental.pallas import tpu_sc as plsc

N = 10000
NP = 10240
E = 320000
D = 128

NC = 2
NS = 16
NW = NC * NS
CH = 128
NCHT = 80
E_PAD = NW * NCHT * CH
ROWS_PT = NP // NS

_MESH = plsc.VectorSubcoreMesh(core_axis_name="c", subcore_axis_name="s")


def _deg_body(dstm, ewm, z_hbm, out_hbm, dst_v, ew_v, deg_sh):
    cid = lax.axis_index("c")
    sid = lax.axis_index("s")
    wid = sid * NC + cid
    pltpu.sync_copy(z_hbm.at[pl.ds(sid * ROWS_PT, ROWS_PT)],
                    deg_sh.at[pl.ds(sid * ROWS_PT, ROWS_PT)])
    pltpu.sync_copy(dstm.at[wid], dst_v)
    pltpu.sync_copy(ewm.at[wid], ew_v)
    plsc.subcore_barrier()

    def chunk(j, carry):
        pltpu.sync_copy(ew_v.at[j], deg_sh.at[dst_v.at[j]], add=True)
        return carry

    lax.fori_loop(0, NCHT, chunk, 0)
    plsc.subcore_barrier()
    pltpu.sync_copy(deg_sh.at[pl.ds(sid * ROWS_PT, ROWS_PT)],
                    out_hbm.at[cid, pl.ds(sid * ROWS_PT, ROWS_PT)])


_deg_kernel = pl.kernel(
    _deg_body, mesh=_MESH,
    out_type=jax.ShapeDtypeStruct((NC, NP), jnp.float32),
    scratch_types=[
        pltpu.VMEM((NCHT, CH), jnp.int32),
        pltpu.VMEM((NCHT, CH), jnp.float32),
        pltpu.VMEM_SHARED((NP,), jnp.float32),
    ],
)


def _agg_body(h_hbm, srcm, dstm, ewm, z_hbm, out_hbm,
              src_v, dst_v, ew_v, acc_sh, rows, sem):
    cid = lax.axis_index("c")
    sid = lax.axis_index("s")
    wid = sid * NC + cid
    pltpu.sync_copy(z_hbm.at[pl.ds(sid * ROWS_PT, ROWS_PT)],
                    acc_sh.at[pl.ds(sid * ROWS_PT, ROWS_PT)])
    pltpu.sync_copy(srcm.at[wid], src_v)
    pltpu.sync_copy(dstm.at[wid], dst_v)
    pltpu.sync_copy(ewm.at[wid], ew_v)
    plsc.subcore_barrier()

    def chunk(j, carry):
        pltpu.async_copy(h_hbm.at[src_v.at[j]], rows, sem).wait()

        def edge_group(g, c2):
            ews = ew_v[j, pl.ds(g * 16, 16)]
            for i in range(16):
                s = ews[i]
                e = g * 16 + i
                for k in range(D // 16):
                    sl = pl.ds(k * 16, 16)
                    rows[e, sl] = rows[e, sl] * s
            return c2

        lax.fori_loop(0, CH // 16, edge_group, 0)
        pltpu.sync_copy(rows, acc_sh.at[dst_v.at[j]], add=True)
        return carry

    lax.fori_loop(0, NCHT, chunk, 0)
    plsc.subcore_barrier()
    pltpu.sync_copy(acc_sh.at[pl.ds(sid * ROWS_PT, ROWS_PT)],
                    out_hbm.at[cid, pl.ds(sid * ROWS_PT, ROWS_PT)])


_agg_kernel = pl.kernel(
    _agg_body, mesh=_MESH,
    out_type=jax.ShapeDtypeStruct((NC, NP, D), jnp.float32),
    scratch_types=[
        pltpu.VMEM((NCHT, CH), jnp.int32),
        pltpu.VMEM((NCHT, CH), jnp.int32),
        pltpu.VMEM((NCHT, CH), jnp.float32),
        pltpu.VMEM_SHARED((NP, D), jnp.float32),
        pltpu.VMEM((CH, D), jnp.float32),
        pltpu.SemaphoreType.DMA,
    ],
)


BM = 1024


def _dis(d0_ref, d1_ref):
    return lax.rsqrt(d0_ref[...] + d1_ref[...] + 1.0)


def _mm1_body(x_ref, w_ref, d0_ref, d1_ref, o_ref):
    dis = _dis(d0_ref, d1_ref)
    o_ref[...] = dis * jnp.dot(x_ref[...], w_ref[...],
                               preferred_element_type=jnp.float32)


def _mid_body(a0_ref, a1_ref, hp_ref, d0_ref, d1_ref, b_ref, w_ref, o_ref):
    dis = _dis(d0_ref, d1_ref)
    t = dis * (a0_ref[...] + a1_ref[...] + hp_ref[...]) + b_ref[...]
    t = jnp.where(t >= 0.0, t, 0.01 * t)
    o_ref[...] = dis * jnp.dot(t, w_ref[...],
                               preferred_element_type=jnp.float32)


def _fin_body(a0_ref, a1_ref, hp_ref, d0_ref, d1_ref, b_ref, o_ref):
    dis = _dis(d0_ref, d1_ref)
    o_ref[...] = dis * (a0_ref[...] + a1_ref[...] + hp_ref[...]) + b_ref[...]


_nd_spec = pl.BlockSpec((BM, D), lambda i: (i, 0))
_d_spec = pl.BlockSpec((BM, 1), lambda i: (i, 0))
_w_spec = pl.BlockSpec((D, D), lambda i: (0, 0))
_b_spec = pl.BlockSpec((1, D), lambda i: (0, 0))
_out_nd = jax.ShapeDtypeStruct((NP, D), jnp.float32)
_grid = (NP // BM,)

_mm1 = pl.pallas_call(
    _mm1_body, grid=_grid,
    in_specs=[_nd_spec, _w_spec, _d_spec, _d_spec],
    out_specs=_nd_spec, out_shape=_out_nd)

_mid = pl.pallas_call(
    _mid_body, grid=_grid,
    in_specs=[_nd_spec, _nd_spec, _nd_spec, _d_spec, _d_spec, _b_spec, _w_spec],
    out_specs=_nd_spec, out_shape=_out_nd)

_fin = pl.pallas_call(
    _fin_body, grid=_grid,
    in_specs=[_nd_spec, _nd_spec, _nd_spec, _d_spec, _d_spec, _b_spec],
    out_specs=_nd_spec, out_shape=_out_nd)


def kernel(x, edge_index, edge_attr, W1, b1, W2, b2, W3, b3):
    src = edge_index[0].astype(jnp.int32)
    dst = edge_index[1].astype(jnp.int32)
    ew = edge_attr.astype(jnp.float32)

    pad = E_PAD - E
    srcm = jnp.concatenate([src, jnp.zeros((pad,), jnp.int32)])
    dstm = jnp.concatenate([dst, jnp.zeros((pad,), jnp.int32)])
    ewm = jnp.concatenate([ew, jnp.zeros((pad,), jnp.float32)])
    srcm = srcm.reshape(NW, NCHT, CH)
    dstm = dstm.reshape(NW, NCHT, CH)
    ewm = ewm.reshape(NW, NCHT, CH)

    xpad = jnp.concatenate([x, jnp.zeros((NP - N, D), jnp.float32)])
    z_n1 = jnp.zeros((NP,), jnp.float32)
    z_nd = jnp.zeros((NP, D), jnp.float32)

    degp = _deg_kernel(dstm, ewm, z_n1)
    d0 = degp[0].reshape(NP, 1)
    d1 = degp[1].reshape(NP, 1)

    b1r = b1.reshape(1, D)
    b2r = b2.reshape(1, D)
    b3r = b3.reshape(1, D)

    hp = _mm1(xpad, W1, d0, d1)
    accp = _agg_kernel(hp, srcm, dstm, ewm, z_nd)
    hp = _mid(accp[0], accp[1], hp, d0, d1, b1r, W2)
    accp = _agg_kernel(hp, srcm, dstm, ewm, z_nd)
    hp = _mid(accp[0], accp[1], hp, d0, d1, b2r, W3)
    accp = _agg_kernel(hp, srcm, dstm, ewm, z_nd)
    return _fin(accp[0], accp[1], hp, d0, d1, b3r)[:N]
